# Initial kernel scaffold; baseline (speedup 1.0000x reference)
import jax
import jax.numpy as jnp
from jax.experimental import pallas as pl


def kernel(x, edge_index, edge_weight_src_to_tgt, edge_weight_tgt_to_src, W_src_to_dst, W_dst_to_src, b_src_to_dst, b_dst_to_src):
    raise NotImplementedError("write your pallas kernel here")


# speedup vs baseline: 3.6991x; 3.6991x over previous
import functools

import jax
import jax.numpy as jnp
from jax import lax
from jax.experimental import pallas as pl
from jax.experimental.pallas import tpu as pltpu
from jax.experimental.pallas import tpu_sc as plsc

N_NODES = 10000
N_PAD = 10240
D = 128
NC = 2
NS = 16
LANES = 16
B = 128
ROWS_PER_TILE = N_PAD // NS


def _sc_aggregate(x, src, dst, w1, w2, nbatches):
    mesh = plsc.VectorSubcoreMesh(
        core_axis_name="c", subcore_axis_name="s", num_cores=NC, num_subcores=NS
    )

    @functools.partial(
        pl.kernel,
        out_type=jax.ShapeDtypeStruct((NC, N_PAD, D), jnp.float32),
        mesh=mesh,
        scratch_types=[
            pltpu.VMEM_SHARED((N_PAD, D), jnp.float32),
            pltpu.VMEM((B,), jnp.int32),
            pltpu.VMEM((B,), jnp.int32),
            pltpu.VMEM((B,), jnp.float32),
            pltpu.VMEM((B, D), jnp.float32),
            pltpu.SemaphoreType.DMA,
        ],
    )
    def k(x_hbm, src_hbm, dst_hbm, w1_hbm, w2_hbm, out_hbm,
          acc_sh, srcv, dstv, wv, rows, sem):
        c = lax.axis_index("c")
        s = lax.axis_index("s")
        row0 = s * ROWS_PER_TILE

        def zrow(j, carry):
            for kk in range(D // LANES):
                rows[j, pl.ds(kk * LANES, LANES)] = jnp.zeros((LANES,), jnp.float32)
            return carry
        lax.fori_loop(0, B, zrow, 0)
        for i in range(ROWS_PER_TILE // B):
            pltpu.sync_copy(rows, acc_sh.at[pl.ds(row0 + i * B, B)])
        plsc.subcore_barrier()

        base = s * (nbatches * B)

        def body(b, carry):
            off = base + b * B
            pltpu.sync_copy(src_hbm.at[pl.ds(off, B)], srcv)
            pltpu.sync_copy(dst_hbm.at[pl.ds(off, B)], dstv)

            @pl.when(c == 0)
            def _():
                pltpu.sync_copy(w1_hbm.at[pl.ds(off, B)], wv)

            @pl.when(c != 0)
            def _():
                pltpu.sync_copy(w2_hbm.at[pl.ds(off, B)], wv)

            pltpu.async_copy(x_hbm.at[srcv], rows, sem).wait()

            def scale(g, carry2):
                wgroup = wv[pl.ds(g * LANES, LANES)]
                for jj in range(LANES):
                    wj = wgroup[jj]
                    j = g * LANES + jj
                    for kk in range(D // LANES):
                        sl = pl.ds(kk * LANES, LANES)
                        rows[j, sl] = rows[j, sl] * wj
                return carry2
            lax.fori_loop(0, B // LANES, scale, 0)

            pltpu.sync_copy(rows, acc_sh.at[dstv], add=True)
            return carry
        lax.fori_loop(0, nbatches, body, 0)

        plsc.subcore_barrier()
        pltpu.sync_copy(acc_sh.at[pl.ds(row0, ROWS_PER_TILE)],
                        out_hbm.at[c, pl.ds(row0, ROWS_PER_TILE)])

    return k(x, src, dst, w1, w2)


def _tc_combine(agg, W1, W2, bias):
    BM = 1000
    grid = (N_NODES // BM,)

    def body(a0, a1, w1, w2, bref, o):
        o[:, :] = (
            jnp.dot(a0[0], w1[:, :], preferred_element_type=jnp.float32)
            + jnp.dot(a1[0], w2[:, :], preferred_element_type=jnp.float32)
            + bref[:, :]
        )

    return pl.pallas_call(
        body,
        grid=grid,
        in_specs=[
            pl.BlockSpec((1, BM, D), lambda i: (0, i, 0)),
            pl.BlockSpec((1, BM, D), lambda i: (1, i, 0)),
            pl.BlockSpec((D, D), lambda i: (0, 0)),
            pl.BlockSpec((D, D), lambda i: (0, 0)),
            pl.BlockSpec((1, D), lambda i: (0, 0)),
        ],
        out_specs=pl.BlockSpec((BM, D), lambda i: (i, 0)),
        out_shape=jax.ShapeDtypeStruct((N_NODES, D), jnp.float32),
    )(agg, agg, W1, W2, bias)


def kernel(x, edge_index, edge_weight_src_to_tgt, edge_weight_tgt_to_src,
           W_src_to_dst, W_dst_to_src, b_src_to_dst, b_dst_to_src):
    E = edge_index.shape[1]
    nbatches = -(-E // (NS * B))
    epad = NS * B * nbatches
    pad = epad - E
    src = jnp.pad(edge_index[0], (0, pad))
    dst = jnp.pad(edge_index[1], (0, pad))
    w1 = jnp.pad(edge_weight_src_to_tgt[:, 0], (0, pad))
    w2 = jnp.pad(edge_weight_tgt_to_src[:, 0], (0, pad))
    agg = _sc_aggregate(x, src, dst, w1, w2, nbatches)
    bias = (b_src_to_dst + b_dst_to_src).reshape(1, D)
    return _tc_combine(agg, W_src_to_dst, W_dst_to_src, bias)
